# baseline (speedup 1.0000x reference)
import functools

import jax
import jax.numpy as jnp
from jax import lax
from jax.experimental import pallas as pl
from jax.experimental.pallas import tpu as pltpu
from jax.experimental.pallas import tpu_sc as plsc

L = 16
NSUB = 16
N = 60000
STRIDE = 3744
PER = 3840
CH = PER // L
SEG = PER + L
NCLS = 300
UNROLL = 4
HI0 = 0x7F800000

_f32 = jnp.float32
_i32 = jnp.int32


def _softplus16(x):
    z = jnp.exp(-jnp.abs(x))
    w = z / (z + _f32(2.0))
    w2 = w * w
    p = _f32(1.0 / 13.0)
    p = _f32(1.0 / 11.0) + w2 * p
    p = _f32(1.0 / 9.0) + w2 * p
    p = _f32(1.0 / 7.0) + w2 * p
    p = _f32(1.0 / 5.0) + w2 * p
    p = _f32(1.0 / 3.0) + w2 * p
    p = _f32(1.0) + w2 * p
    return jnp.maximum(x, _f32(0.0)) + _f32(2.0) * w * p


@functools.cache
def _build():
    mesh = plsc.VectorSubcoreMesh(core_axis_name="c", subcore_axis_name="s")

    @functools.partial(
        pl.kernel,
        out_type=jax.ShapeDtypeStruct((L,), _f32),
        mesh=mesh,
        compiler_params=pltpu.CompilerParams(needs_layout_passes=False),
        scratch_types=[
            pltpu.VMEM((2 * PER,), _f32),
            pltpu.VMEM((2 * PER,), _i32),
            pltpu.VMEM((SEG,), _f32),
            pltpu.VMEM((NSUB * SEG + L,), _f32),
            pltpu.VMEM((NSUB * L,), _f32),
            pltpu.VMEM((NSUB * L,), _i32),
            pltpu.VMEM((NSUB * L,), _i32),
            pltpu.VMEM((L,), _f32),
            pltpu.VMEM((L,), _i32),
            pltpu.VMEM((L,), _i32),
            pltpu.VMEM((L,), _f32),
            pltpu.VMEM_SHARED((NSUB * SEG,), _f32),
            pltpu.VMEM_SHARED((NSUB * L,), _f32),
            pltpu.VMEM_SHARED((NSUB * L,), _i32),
            pltpu.VMEM_SHARED((NSUB * L,), _i32),
            pltpu.SemaphoreType.DMA,
            pltpu.SemaphoreType.DMA,
        ],
    )
    def k(x_hbm, y_hbm, out_hbm,
          x_v, y_v, bgbuf, dense, meta_fg_v, meta_nfg_v,
          meta_off_v, stage_fg, stage_nfg, stage_off, outbuf,
          sh_bg, sh_fg, sh_nfg, sh_off, sem0, sem1):
        cid = lax.axis_index("c")
        sid = lax.axis_index("s")

        @pl.when(cid == 0)
        def _core0():
            zf = jnp.zeros((L,), _f32)
            zi = jnp.zeros((L,), _i32)
            lane = lax.broadcasted_iota(_i32, (L,), 0)
            lane2 = lane + lane

            base = sid * STRIDE
            c0 = pltpu.async_copy(x_hbm.at[pl.ds(2 * base, 2 * PER)],
                                  x_v, sem0)
            c1 = pltpu.async_copy(y_hbm.at[pl.ds(2 * base, 2 * PER)],
                                  y_v, sem1)
            limit = jnp.where(sid == NSUB - 1, _i32(N),
                              (sid + _i32(1)) * _i32(STRIDE))
            rel_limit_v = (zi + limit) - base
            c0.wait()
            c1.wait()

            def body(i, carry):
                off_v, fg_acc, nfg_acc = carry
                for u in range(UNROLL):
                    cidx = i * UNROLL + u
                    idx0 = lane2 + cidx * (2 * L)
                    x0 = plsc.load_gather(x_v, [idx0])
                    x1 = plsc.load_gather(x_v, [idx0 + _i32(1)])
                    lb = plsc.load_gather(y_v, [idx0])
                    valid = (lane + cidx * L) < rel_limit_v
                    dd = x1 - x0
                    is_fg = (lb == 1) & valid
                    is_bg = (lb == 0) & valid
                    loss = _softplus16(jnp.where(is_fg, -dd, dd))
                    fg_acc = fg_acc + jnp.where(is_fg, loss, _f32(0.0))
                    nfg_acc = nfg_acc + jnp.where(is_fg, _i32(1), _i32(0))
                    bg_i = jnp.where(is_bg, _i32(1), _i32(0))
                    pos = off_v + lax.cumsum(bg_i, axis=0) - _i32(1)
                    plsc.store_scatter(bgbuf, [pos], loss, mask=is_bg)
                    off_v = off_v + plsc.all_reduce_population_count(is_bg)
                return off_v, fg_acc, nfg_acc

            off_v, fg_acc, nfg_acc = lax.fori_loop(
                _i32(0), _i32(CH // UNROLL), body, (zi, zf, zi))
            off = jnp.max(off_v)
            plsc.store_scatter(bgbuf, [off + lane],
                               jnp.full((L,), -jnp.inf, _f32))

            stage_fg[...] = fg_acc
            stage_nfg[...] = nfg_acc
            stage_off[...] = zi + off
            pltpu.sync_copy(bgbuf, sh_bg.at[pl.ds(sid * SEG, SEG)])
            pltpu.sync_copy(stage_fg, sh_fg.at[pl.ds(sid * L, L)])
            pltpu.sync_copy(stage_nfg, sh_nfg.at[pl.ds(sid * L, L)])
            pltpu.sync_copy(stage_off, sh_off.at[pl.ds(sid * L, L)])
            plsc.subcore_barrier()

            @pl.when(sid == 0)
            def _merge():
                pltpu.sync_copy(sh_fg, meta_fg_v)
                pltpu.sync_copy(sh_nfg, meta_nfg_v)
                pltpu.sync_copy(sh_off, meta_off_v)

                def red(w_, carry):
                    fg_v, nfg_v = carry
                    slw = pl.ds(w_ * L, L)
                    return fg_v + meta_fg_v[slw], nfg_v + meta_nfg_v[slw]

                fg_v, nfg_v = lax.fori_loop(_i32(0), _i32(NSUB), red, (zf, zi))
                fg_sum = jnp.sum(fg_v)
                n_fg = jnp.sum(nfg_v, dtype=_i32)

                def gather_w(w_, carry):
                    g, nbg = carry
                    offw = jnp.max(meta_off_v[pl.ds(w_ * L, L)])
                    nch = lax.shift_right_logical(offw + _i32(L - 1), _i32(4))

                    def cp(j, gg):
                        pltpu.sync_copy(
                            sh_bg.at[pl.ds(w_ * SEG + j * L, L)],
                            dense.at[pl.ds(gg * L, L)])
                        return gg + _i32(1)

                    g = lax.fori_loop(_i32(0), nch, cp, g)
                    return g, nbg + offw

                G, n_bg = lax.fori_loop(_i32(0), _i32(NSUB), gather_w,
                                        (_i32(0), _i32(0)))
                dense[pl.ds(G * L, L)] = jnp.full((L,), -jnp.inf, _f32)
                G2 = lax.shift_right_logical(G + _i32(1), _i32(1))
                K = _i32(NCLS) - n_fg

                def bs(_, carry):
                    lo, hi = carry
                    mid = lo + lax.shift_right_logical(hi - lo, _i32(1))
                    tv = plsc.bitcast(zi + mid, _f32)

                    def cb(j, acc):
                        va = dense[pl.ds(j * (2 * L), L)]
                        vb = dense[pl.ds(j * (2 * L) + L, L)]
                        return (acc + jnp.where(va >= tv, _i32(1), _i32(0))
                                + jnp.where(vb >= tv, _i32(1), _i32(0)))

                    c = jnp.sum(lax.fori_loop(_i32(0), G2, cb, zi),
                                dtype=_i32)
                    pred = c >= K
                    return (jnp.where(pred, mid, lo),
                            jnp.where(pred, hi, mid))

                lo, _hi = lax.fori_loop(_i32(0), _i32(31), bs,
                                        (_i32(0), _i32(HI0)))
                tv = plsc.bitcast(zi + lo, _f32)

                def fin(j, carry):
                    cv, sv = carry
                    va = dense[pl.ds(j * (2 * L), L)]
                    vb = dense[pl.ds(j * (2 * L) + L, L)]
                    ma = va > tv
                    mb = vb > tv
                    return (cv + jnp.where(ma, _i32(1), _i32(0))
                            + jnp.where(mb, _i32(1), _i32(0)),
                            sv + jnp.where(ma, va, _f32(0.0))
                            + jnp.where(mb, vb, _f32(0.0)))

                cv, sv = lax.fori_loop(_i32(0), G2, fin, (zi, zf))
                c_gt = jnp.sum(cv, dtype=_i32)
                s_gt = jnp.sum(sv)
                t_s = jnp.max(tv)
                bg_main = s_gt + (K - c_gt).astype(_f32) * t_s
                bg_sum = jnp.where(
                    K <= _i32(0), _f32(0.0),
                    jnp.where(K > n_bg, _f32(-jnp.inf), bg_main))
                outbuf[...] = (zf + (fg_sum + bg_sum)) / (zf + _f32(NCLS))
                pltpu.sync_copy(outbuf, out_hbm)

    return k


def kernel(输入, 标签):
    x = 输入.reshape(2 * N)
    y = lax.bitcast_convert_type(标签[0, 0], _i32).reshape(2 * N)
    out = _build()(x, y)
    return out[0]


# speedup vs baseline: 3.1443x; 3.1443x over previous
import functools

import jax
import jax.numpy as jnp
from jax import lax
from jax.experimental import pallas as pl
from jax.experimental.pallas import tpu as pltpu
from jax.experimental.pallas import tpu_sc as plsc

L = 16
NSUB = 16
PER = 3840
NPAD = NSUB * PER
CH = PER // L
SEG = PER + L
NCLS = 300
UNROLL = 4
HI0 = 0x7F800000
CAP1 = 512
CAP2 = 4096

_f32 = jnp.float32
_i32 = jnp.int32


def _softplus16(x):
    z = jnp.exp(-jnp.abs(x))
    w = z / (z + _f32(2.0))
    w2 = w * w
    p = _f32(1.0 / 13.0)
    p = _f32(1.0 / 11.0) + w2 * p
    p = _f32(1.0 / 9.0) + w2 * p
    p = _f32(1.0 / 7.0) + w2 * p
    p = _f32(1.0 / 5.0) + w2 * p
    p = _f32(1.0 / 3.0) + w2 * p
    p = _f32(1.0) + w2 * p
    return jnp.maximum(x, _f32(0.0)) + _f32(2.0) * w * p


@functools.cache
def _build():
    mesh = plsc.VectorSubcoreMesh(core_axis_name="c", subcore_axis_name="s")

    @functools.partial(
        pl.kernel,
        out_type=jax.ShapeDtypeStruct((L,), _f32),
        mesh=mesh,
        compiler_params=pltpu.CompilerParams(needs_layout_passes=False),
        scratch_types=[
            pltpu.VMEM((PER,), _f32),
            pltpu.VMEM((PER,), _f32),
            pltpu.VMEM((PER,), _i32),
            pltpu.VMEM((SEG,), _f32),
            pltpu.VMEM((NSUB * SEG + L,), _f32),
            pltpu.VMEM((NSUB * L,), _f32),
            pltpu.VMEM((L,), _f32),
            pltpu.VMEM((L,), _f32),
            pltpu.SMEM((4,), _i32),
            pltpu.VMEM_SHARED((NSUB * SEG,), _f32),
            pltpu.VMEM_SHARED((NSUB * L,), _f32),
            pltpu.SemaphoreType.DMA,
            pltpu.SemaphoreType.DMA,
            pltpu.SemaphoreType.DMA,
        ],
    )
    def k(l0_hbm, l1_hbm, lab_hbm, out_hbm,
          l0_v, l1_v, lab_v, bgbuf, dense, meta_fg_v, stage_fg, outbuf,
          counters, sh_bg, sh_fg, sem0, sem1, sem2):
        cid = lax.axis_index("c")
        sid = lax.axis_index("s")

        @pl.when(cid == 0)
        def _core0():
            zf = jnp.zeros((L,), _f32)
            zi = jnp.zeros((L,), _i32)
            lane = lax.broadcasted_iota(_i32, (L,), 0)

            base = sid * PER
            c0 = pltpu.async_copy(l0_hbm.at[pl.ds(base, PER)], l0_v, sem0)
            c1 = pltpu.async_copy(l1_hbm.at[pl.ds(base, PER)], l1_v, sem1)
            c2 = pltpu.async_copy(lab_hbm.at[pl.ds(base, PER)], lab_v, sem2)

            @pl.when(sid == 0)
            def _init():
                counters[_i32(0)] = _i32(0)
                counters[_i32(1)] = _i32(0)
                counters[_i32(2)] = _i32(0)

            c0.wait()
            c1.wait()
            c2.wait()
            plsc.subcore_barrier()

            def body(i, carry):
                off_v, fg_acc, nfg_acc = carry
                for u in range(UNROLL):
                    sl = pl.ds((i * UNROLL + u) * L, L)
                    x0 = l0_v[sl]
                    x1 = l1_v[sl]
                    lb = lab_v[sl]
                    dd = x1 - x0
                    is_fg = lb == 1
                    is_bg = lb == 0
                    loss = _softplus16(jnp.where(is_fg, -dd, dd))
                    fg_acc = fg_acc + jnp.where(is_fg, loss, _f32(0.0))
                    nfg_acc = nfg_acc + jnp.where(is_fg, _i32(1), _i32(0))
                    bg_i = jnp.where(is_bg, _i32(1), _i32(0))
                    pos = off_v + lax.cumsum(bg_i, axis=0) - _i32(1)
                    plsc.store_scatter(bgbuf, [pos], loss, mask=is_bg)
                    off_v = off_v + plsc.all_reduce_population_count(is_bg)
                return off_v, fg_acc, nfg_acc

            off_v, fg_acc, nfg_acc = lax.fori_loop(
                _i32(0), _i32(CH // UNROLL), body, (zi, zf, zi))
            off = jnp.max(off_v)
            plsc.store_scatter(bgbuf, [off + lane],
                               jnp.full((L,), -jnp.inf, _f32))

            nch = lax.shift_right_logical(off + _i32(L - 1), _i32(4))
            words = nch * _i32(L)
            gbase = plsc.fetch_and_add(counters.at[_i32(0)], words, subcore_id=_i32(0))
            plsc.fetch_and_add(counters.at[_i32(1)], off, subcore_id=_i32(0))
            nfg_me = jnp.sum(nfg_acc, dtype=_i32)
            plsc.fetch_and_add(counters.at[_i32(2)], nfg_me, subcore_id=_i32(0))

            def cp(j, _):
                pltpu.sync_copy(bgbuf.at[pl.ds(j * L, L)],
                                sh_bg.at[pl.ds(pl.multiple_of(gbase + j * L, L), L)])
                return _

            lax.fori_loop(_i32(0), nch, cp, _i32(0))
            stage_fg[...] = fg_acc
            pltpu.sync_copy(stage_fg, sh_fg.at[pl.ds(sid * L, L)])
            plsc.subcore_barrier()

            @pl.when(sid == 0)
            def _merge():
                gw = counters[_i32(0)]
                n_bg = counters[_i32(1)]
                n_fg = counters[_i32(2)]
                G = lax.shift_right_logical(gw, _i32(4))
                pltpu.sync_copy(sh_fg, meta_fg_v)

                @pl.when(gw <= CAP1)
                def _small():
                    pltpu.sync_copy(sh_bg.at[pl.ds(0, CAP1)],
                                    dense.at[pl.ds(0, CAP1)])

                @pl.when((gw > CAP1) & (gw <= CAP2))
                def _mid():
                    pltpu.sync_copy(sh_bg.at[pl.ds(0, CAP2)],
                                    dense.at[pl.ds(0, CAP2)])

                @pl.when(gw > CAP2)
                def _full():
                    pltpu.sync_copy(sh_bg, dense.at[pl.ds(0, NSUB * SEG)])

                def red(w_, fg_v):
                    return fg_v + meta_fg_v[pl.ds(w_ * L, L)]

                fg_v = lax.fori_loop(_i32(0), _i32(NSUB), red, zf)
                fg_sum = jnp.sum(fg_v)

                dense[pl.ds(G * L, L)] = jnp.full((L,), -jnp.inf, _f32)
                G2 = lax.shift_right_logical(G + _i32(1), _i32(1))
                K = _i32(NCLS) - n_fg

                def bs(_, carry):
                    lo, hi = carry
                    mid = lo + lax.shift_right_logical(hi - lo, _i32(1))
                    tv = plsc.bitcast(zi + mid, _f32)

                    def cb(j, acc):
                        va = dense[pl.ds(j * (2 * L), L)]
                        vb = dense[pl.ds(j * (2 * L) + L, L)]
                        return (acc + jnp.where(va >= tv, _i32(1), _i32(0))
                                + jnp.where(vb >= tv, _i32(1), _i32(0)))

                    c = jnp.sum(lax.fori_loop(_i32(0), G2, cb, zi),
                                dtype=_i32)
                    pred = c >= K
                    return (jnp.where(pred, mid, lo),
                            jnp.where(pred, hi, mid))

                lo, _hi = lax.fori_loop(_i32(0), _i32(31), bs,
                                        (_i32(0), _i32(HI0)))
                tv = plsc.bitcast(zi + lo, _f32)

                def fin(j, carry):
                    cv, sv = carry
                    va = dense[pl.ds(j * (2 * L), L)]
                    vb = dense[pl.ds(j * (2 * L) + L, L)]
                    ma = va > tv
                    mb = vb > tv
                    return (cv + jnp.where(ma, _i32(1), _i32(0))
                            + jnp.where(mb, _i32(1), _i32(0)),
                            sv + jnp.where(ma, va, _f32(0.0))
                            + jnp.where(mb, vb, _f32(0.0)))

                cv, sv = lax.fori_loop(_i32(0), G2, fin, (zi, zf))
                c_gt = jnp.sum(cv, dtype=_i32)
                s_gt = jnp.sum(sv)
                t_s = jnp.max(tv)
                bg_main = s_gt + (K - c_gt).astype(_f32) * t_s
                bg_sum = jnp.where(
                    K <= _i32(0), _f32(0.0),
                    jnp.where(K > n_bg, _f32(-jnp.inf), bg_main))
                outbuf[...] = (zf + (fg_sum + bg_sum)) / (zf + _f32(NCLS))
                pltpu.sync_copy(outbuf, out_hbm)

    return k


def kernel(输入, 标签):
    logits = 输入[0]
    labels = 标签[0, 0].astype(_i32)
    n = logits.shape[0]
    pad = NPAD - n
    l0 = jnp.concatenate([logits[:, 0], jnp.zeros((pad,), _f32)])
    l1 = jnp.concatenate([logits[:, 1], jnp.zeros((pad,), _f32)])
    lab = jnp.concatenate([labels, jnp.full((pad,), 2, _i32)])
    out = _build()(l0, l1, lab)
    return out[0]
